# baseline (speedup 1.0000x reference)
import functools

import jax
import jax.numpy as jnp
from jax import lax
from jax.experimental import pallas as pl
from jax.experimental.pallas import tpu as pltpu
from jax.experimental.pallas import tpu_sc as plsc

_B = 4096
_D = 64
_NC = 2
_NS = 16
_NW = _NC * _NS
_BPW = _B // _NW
_L = 16


_GATHER_DN = lax.GatherDimensionNumbers(
    offset_dims=(), collapsed_slice_dims=(0,), start_index_map=(0,))


def _lane_shuffle(v, idx):
    return lax.gather(v, idx[:, None], _GATHER_DN, (1,),
                      mode=lax.GatherScatterMode.PROMISE_IN_BOUNDS)


def _score_block(rows_v, b, out_v):
    lane = lax.iota(jnp.int32, _L)
    perms = [lane ^ m for m in (8, 4, 2, 1)]

    def body(g, carry):
        vec = jnp.zeros((_L,), jnp.float32)
        for j in range(_L):
            i = g * _L + j
            acc = jnp.zeros((_L,), jnp.float32)
            for c in range(_D // _L):
                sl = pl.ds(c * _L, _L)
                v = (rows_v[4 * b + 0, i, sl]
                     + rows_v[4 * b + 1, i, sl]
                     + rows_v[4 * b + 3, i, sl]
                     - rows_v[4 * b + 2, i, sl])
                acc = acc + v * v
            for p in perms:
                acc = acc + _lane_shuffle(acc, p)
            vec = jnp.where(lane == j, acc, vec)
        out_v[b, pl.ds(g * _L, _L)] = vec
        return carry

    lax.fori_loop(0, _BPW // _L, body, 0)


@functools.partial(
    pl.kernel,
    out_type=[
        jax.ShapeDtypeStruct((_B,), jnp.float32),
        jax.ShapeDtypeStruct((_B,), jnp.float32),
    ],
    mesh=plsc.VectorSubcoreMesh(core_axis_name="c", subcore_axis_name="s"),
    compiler_params=pltpu.CompilerParams(use_tc_tiling_on_sc=False),
    scratch_types=[
        pltpu.VMEM((8, _BPW), jnp.int32),
        pltpu.VMEM((8, _BPW, _D), jnp.float32),
        pltpu.VMEM((2, _BPW), jnp.float32),
        pltpu.SemaphoreType.DMA,
        pltpu.SemaphoreType.DMA,
    ],
)
def _ttranse_sc(idx_hbm, entity_hbm, relation_hbm, out_correct, out_corrupt,
                idx_v, rows_v, out_v, sem0, sem1):
    wid = lax.axis_index("s") * _NC + lax.axis_index("c")
    base = wid * _BPW

    pltpu.sync_copy(idx_hbm.at[:, pl.ds(base, _BPW)], idx_v)

    tables = (entity_hbm, relation_hbm, entity_hbm, relation_hbm)
    sems = (sem0, sem1)
    handles = []
    for b in range(2):
        for j in range(4):
            handles.append(
                pltpu.async_copy(
                    tables[j].at[idx_v.at[4 * b + j]],
                    rows_v.at[4 * b + j],
                    sems[b],
                )
            )

    for b in range(2):
        for j in range(4):
            handles[4 * b + j].wait()
        _score_block(rows_v, b, out_v)

    pltpu.sync_copy(out_v.at[0], out_correct.at[pl.ds(base, _BPW)])
    pltpu.sync_copy(out_v.at[1], out_corrupt.at[pl.ds(base, _BPW)])


def kernel(batch, corrupt_batch, entity_emb, relation_emb):
    idx = jnp.concatenate([batch.T, corrupt_batch.T], axis=0)
    correct, corrupt = _ttranse_sc(idx, entity_emb, relation_emb)
    return (correct, corrupt)
